# Initial kernel scaffold; baseline (speedup 1.0000x reference)
import functools

import jax
import jax.numpy as jnp
from jax import lax
from jax.experimental import pallas as pl
from jax.experimental.pallas import tpu as pltpu
from jax.experimental.pallas import tpu_sc as plsc

N = 10000
NPAD = 10240
E = 320000
D = 128
G = 64
NWORK = 32
CHUNK = 128
ROWS_W = 79
EPAD = NWORK * ROWS_W * CHUNK
STRIPE = NPAD // 16


def _sc_body(with_counts, *refs):
    if with_counts:
        (y_hbm, src_hbm, dst_hbm, zacc_hbm, zcnt_hbm,
         agg_out, cnt_out,
         srcv, dstv, rows, ones_v, gsem) = refs
    else:
        (y_hbm, src_hbm, dst_hbm, zacc_hbm,
         agg_out,
         srcv, dstv, rows, gsem) = refs
        zcnt_hbm = cnt_out = ones_v = None

    cid = lax.axis_index("c")
    sid = lax.axis_index("s")
    wid = cid * 16 + sid
    base = sid * STRIPE

    def _scoped(acc_sh, cnt_sh=None):
        pltpu.sync_copy(zacc_hbm.at[pl.ds(base, STRIPE)],
                        acc_sh.at[pl.ds(base, STRIPE)])
        if with_counts:
            pltpu.sync_copy(zcnt_hbm.at[pl.ds(base, STRIPE)],
                            cnt_sh.at[pl.ds(base, STRIPE)])
            ones = jnp.ones((16,), jnp.float32)

            def fill(i, _):
                ones_v[i, :] = ones
                return 0
            lax.fori_loop(0, CHUNK, fill, 0)
        plsc.subcore_barrier()

        pltpu.sync_copy(src_hbm.at[wid], srcv)
        pltpu.sync_copy(dst_hbm.at[wid], dstv)

        def step(j, _):
            pltpu.async_copy(y_hbm.at[srcv.at[j]], rows, gsem).wait()
            pltpu.sync_copy(rows, acc_sh.at[dstv.at[j]], add=True)
            if with_counts:
                pltpu.sync_copy(ones_v, cnt_sh.at[dstv.at[j]], add=True)
            return 0
        lax.fori_loop(0, ROWS_W, step, 0)

        plsc.subcore_barrier()
        pltpu.sync_copy(acc_sh.at[pl.ds(base, STRIPE)],
                        agg_out.at[cid].at[pl.ds(base, STRIPE)])
        if with_counts:
            pltpu.sync_copy(cnt_sh.at[pl.ds(base, STRIPE)],
                            cnt_out.at[cid].at[pl.ds(base, STRIPE)])

    if with_counts:
        pl.run_scoped(
            _scoped,
            pltpu.VMEM_SHARED((NPAD, D), jnp.float32),
            pltpu.VMEM_SHARED((NPAD, 16), jnp.float32),
        )
    else:
        pl.run_scoped(
            _scoped,
            pltpu.VMEM_SHARED((NPAD, D), jnp.float32),
        )


def _make_sc_kernel(with_counts):
    mesh = plsc.VectorSubcoreMesh(core_axis_name="c", subcore_axis_name="s")
    out_type = [jax.ShapeDtypeStruct((2, NPAD, D), jnp.float32)]
    scratch = [
        pltpu.VMEM((ROWS_W, CHUNK), jnp.int32),
        pltpu.VMEM((ROWS_W, CHUNK), jnp.int32),
        pltpu.VMEM((CHUNK, D), jnp.float32),
    ]
    if with_counts:
        out_type.append(jax.ShapeDtypeStruct((2, NPAD, 16), jnp.float32))
        scratch.append(pltpu.VMEM((CHUNK, 16), jnp.float32))
    scratch.append(pltpu.SemaphoreType.DMA)
    return pl.kernel(
        functools.partial(_sc_body, with_counts),
        out_type=tuple(out_type),
        mesh=mesh,
        scratch_types=tuple(scratch),
    )


BLK = 1024
GRID = NPAD // BLK


def _mm_body(x_ref, w_ref, b_ref, y_ref, z_ref):
    out = jnp.dot(x_ref[...], w_ref[...], preferred_element_type=jnp.float32)
    y_ref[...] = out[:, :D]
    z_ref[...] = out[:, D:] + b_ref[...]


def _fuse_body(p0, p1, c0, c1, z, w_ref, b_ref, y_ref, z_ref):
    cnt = c0[:, 0:1] + c1[:, 0:1]
    inv = 1.0 / jnp.maximum(cnt, 1.0)
    h = jnp.maximum((p0[...] + p1[...]) * inv + z[...], 0.0)
    out = jnp.dot(h, w_ref[...], preferred_element_type=jnp.float32)
    y_ref[...] = out[:, :D]
    z_ref[...] = out[:, D:] + b_ref[...]


def _pool_body(p0, p1, c0, c1, z, wf_ref, batch_ref, bf_ref,
               sums_ref, cnts_ref, out_ref):
    i = pl.program_id(0)
    cnt = c0[:, 0:1] + c1[:, 0:1]
    inv = 1.0 / jnp.maximum(cnt, 1.0)
    h = jnp.maximum((p0[...] + p1[...]) * inv + z[...], 0.0)
    q = jnp.sum(h * wf_ref[...], axis=1, keepdims=True)
    gids = lax.broadcasted_iota(jnp.int32, (BLK, G), 1)
    onehot = (batch_ref[...] == gids).astype(jnp.float32)

    @pl.when(i == 0)
    def _():
        sums_ref[...] = jnp.zeros((1, G), jnp.float32)
        cnts_ref[...] = jnp.zeros((1, G), jnp.float32)

    sums_ref[...] += jnp.sum(onehot * q, axis=0, keepdims=True)
    cnts_ref[...] += jnp.sum(onehot, axis=0, keepdims=True)

    @pl.when(i == GRID - 1)
    def _():
        out_ref[...] = (sums_ref[...] / jnp.maximum(cnts_ref[...], 1.0)
                        + bf_ref[...])


def _row_spec(last):
    return pl.BlockSpec((BLK, last), lambda i: (i, 0))


def _full_spec(shape):
    return pl.BlockSpec(shape, lambda i: (0,) * len(shape))


_mm = pl.pallas_call(
    _mm_body,
    grid=(GRID,),
    in_specs=[_row_spec(D), _full_spec((D, 2 * D)), _full_spec((1, D))],
    out_specs=[_row_spec(D), _row_spec(D)],
    out_shape=[jax.ShapeDtypeStruct((NPAD, D), jnp.float32)] * 2,
)

_fuse = pl.pallas_call(
    _fuse_body,
    grid=(GRID,),
    in_specs=[_row_spec(D), _row_spec(D), _row_spec(16), _row_spec(16),
              _row_spec(D), _full_spec((D, 2 * D)), _full_spec((1, D))],
    out_specs=[_row_spec(D), _row_spec(D)],
    out_shape=[jax.ShapeDtypeStruct((NPAD, D), jnp.float32)] * 2,
)

_pool = pl.pallas_call(
    _pool_body,
    grid=(GRID,),
    in_specs=[_row_spec(D), _row_spec(D), _row_spec(16), _row_spec(16),
              _row_spec(D), _full_spec((1, D)), _row_spec(1),
              _full_spec((1, 1))],
    out_specs=[_full_spec((1, G))] * 3,
    out_shape=[jax.ShapeDtypeStruct((1, G), jnp.float32)] * 3,
)

_sc_layer1 = _make_sc_kernel(True)
_sc_layer2 = _make_sc_kernel(False)


@jax.jit
def kernel(x, edge_index, batch, Wl1, bl1, Wr1, Wl2, bl2, Wr2, Wf, bf):
    src = edge_index[0].astype(jnp.int32)
    dst = edge_index[1].astype(jnp.int32)
    src3 = jnp.pad(src, (0, EPAD - E)).reshape(NWORK, ROWS_W, CHUNK)
    dst3 = jnp.pad(dst, (0, EPAD - E), constant_values=N).reshape(
        NWORK, ROWS_W, CHUNK)
    xp = jnp.pad(x, ((0, NPAD - N), (0, 0)))
    batch_p = jnp.pad(batch.astype(jnp.int32), (0, NPAD - N),
                      constant_values=G).reshape(NPAD, 1)
    w1 = jnp.concatenate([Wl1.T, Wr1.T], axis=1)
    w2 = jnp.concatenate([Wl2.T, Wr2.T], axis=1)
    b1 = bl1.reshape(1, D)
    b2 = bl2.reshape(1, D)
    wf_row = Wf.reshape(1, D)
    bf2 = bf.reshape(1, 1)
    zacc = jnp.zeros((NPAD, D), jnp.float32)
    zcnt = jnp.zeros((NPAD, 16), jnp.float32)

    y1, z1 = _mm(xp, w1, b1)
    aggp1, cntp = _sc_layer1(y1, src3, dst3, zacc, zcnt)
    y2, z2 = _fuse(aggp1[0], aggp1[1], cntp[0], cntp[1], z1, w2, b2)
    (aggp2,) = _sc_layer2(y2, src3, dst3, zacc)
    _, _, out = _pool(aggp2[0], aggp2[1], cntp[0], cntp[1], z2,
                      wf_row, batch_p, bf2)
    return out.reshape(G, 1)


# speedup vs baseline: 3.1995x; 3.1995x over previous
import functools

import jax
import jax.numpy as jnp
from jax import lax
from jax.experimental import pallas as pl
from jax.experimental.pallas import tpu as pltpu
from jax.experimental.pallas import tpu_sc as plsc

N = 10000
NPAD = 10240
E = 320000
D = 128
G = 64
NWORK = 32
CHUNK = 128
ROWS_W = 80
GS = 8
EPAD = NWORK * ROWS_W * CHUNK
STRIPE = NPAD // 16


def _fill_rows(rows, val):
    v = jnp.full((16,), val, jnp.float32)
    lanes = D // 16

    def body(i, _):
        rows[i // lanes, pl.ds((i % lanes) * 16, 16)] = v
        return 0
    lax.fori_loop(0, CHUNK * lanes, body, 0)


def _zero_stripe(rows, acc_sh, base):
    _fill_rows(rows, 0.0)

    def body(k, _):
        pltpu.sync_copy(rows, acc_sh.at[pl.ds(base + k * CHUNK, CHUNK)])
        return 0
    lax.fori_loop(0, STRIPE // CHUNK, body, 0)


def _copy_stripe_out(rows, acc_sh, base, out, obase):
    def body(k, _):
        pltpu.sync_copy(acc_sh.at[pl.ds(base + k * CHUNK, CHUNK)], rows)
        pltpu.sync_copy(rows, out.at[pl.ds(obase + k * CHUNK, CHUNK)])
        return 0
    lax.fori_loop(0, STRIPE // CHUNK, body, 0)


def _sc_body(with_gather, *refs):
    if with_gather:
        (y_hbm, src_hbm, dst_hbm, agg_out,
         srcv, dstv, rows, acc_sh, gsem) = refs
    else:
        (src_hbm, dst_hbm, agg_out,
         srcv, dstv, rows, acc_sh, gsem) = refs

    cid = lax.axis_index("c")
    sid = lax.axis_index("s")
    wid = cid * 16 + sid
    base = sid * STRIPE

    _zero_stripe(rows, acc_sh, base)
    if not with_gather:
        _fill_rows(rows, 1.0)
    plsc.subcore_barrier()

    def group(g, _):
        row0 = wid * ROWS_W + g * GS
        if with_gather:
            pltpu.sync_copy(src_hbm.at[pl.ds(row0, GS)], srcv)
        pltpu.sync_copy(dst_hbm.at[pl.ds(row0, GS)], dstv)

        def step(j, _):
            if with_gather:
                pltpu.async_copy(y_hbm.at[srcv.at[j]], rows, gsem).wait()
            pltpu.sync_copy(rows, acc_sh.at[dstv.at[j]], add=True)
            return 0
        lax.fori_loop(0, GS, step, 0)
        return 0
    lax.fori_loop(0, ROWS_W // GS, group, 0)

    plsc.subcore_barrier()
    _copy_stripe_out(rows, acc_sh, base, agg_out, cid * NPAD + base)


@functools.lru_cache(maxsize=None)
def _make_sc_kernel(with_gather):
    mesh = plsc.VectorSubcoreMesh(core_axis_name="c", subcore_axis_name="s",
                                  num_cores=2, num_subcores=16)
    return pl.kernel(
        functools.partial(_sc_body, with_gather),
        out_type=(jax.ShapeDtypeStruct((2 * NPAD, D), jnp.float32),),
        mesh=mesh,
        scratch_types=(
            pltpu.VMEM((GS, CHUNK), jnp.int32),
            pltpu.VMEM((GS, CHUNK), jnp.int32),
            pltpu.VMEM((CHUNK, D), jnp.float32),
            pltpu.VMEM_SHARED((NPAD, D), jnp.float32),
            pltpu.SemaphoreType.DMA,
        ),
    )


BLK = 1024
GRID = NPAD // BLK


def _bdot(a, b):
    return jnp.dot(a.astype(jnp.bfloat16), b.astype(jnp.bfloat16),
                   preferred_element_type=jnp.float32)


def _layer_body(p0, p1, c0, c1, x_ref, wl_ref, bl_ref, wr_ref, h_ref):
    cnt = c0[:, 0:1] + c1[:, 0:1]
    mean = (p0[...] + p1[...]) / jnp.maximum(cnt, 1.0)
    h = _bdot(mean, wl_ref[...]) + bl_ref[...] + _bdot(x_ref[...], wr_ref[...])
    h_ref[...] = jnp.maximum(h, 0.0)


def _final_body(p0, p1, c0, c1, x_ref, wl_ref, bl_ref, wr_ref, wf_ref,
                batch_ref, bf_ref, sums_ref, cnts_ref, out_ref):
    i = pl.program_id(0)
    cnt = c0[:, 0:1] + c1[:, 0:1]
    mean = (p0[...] + p1[...]) / jnp.maximum(cnt, 1.0)
    h = _bdot(mean, wl_ref[...]) + bl_ref[...] + _bdot(x_ref[...], wr_ref[...])
    h = jnp.maximum(h, 0.0)

    gids = lax.broadcasted_iota(jnp.int32, (BLK, G), 1)
    onehot = (batch_ref[...] == gids).astype(jnp.float32)
    dn = (((0,), (0,)), ((), ()))
    s = lax.dot_general(onehot, h, dn, precision='highest',
                        preferred_element_type=jnp.float32)
    c = lax.dot_general(onehot, jnp.ones((BLK, D), jnp.float32), dn,
                        precision='highest',
                        preferred_element_type=jnp.float32)

    @pl.when(i == 0)
    def _():
        sums_ref[...] = jnp.zeros((G, D), jnp.float32)
        cnts_ref[...] = jnp.zeros((G, D), jnp.float32)

    sums_ref[...] += s
    cnts_ref[...] += c

    @pl.when(i == GRID - 1)
    def _():
        pooled = sums_ref[...] / jnp.maximum(cnts_ref[...], 1.0)
        pb = pooled.astype(jnp.bfloat16).astype(jnp.float32)
        wb = wf_ref[...].astype(jnp.bfloat16).astype(jnp.float32)
        out_ref[...] = jnp.sum(pb * wb, axis=1, keepdims=True) + bf_ref[...]


def _row_spec(last):
    return pl.BlockSpec((BLK, last), lambda i: (i, 0))


def _full_spec(shape):
    return pl.BlockSpec(shape, lambda i: (0,) * len(shape))


_layer = pl.pallas_call(
    _layer_body,
    grid=(GRID,),
    in_specs=[_row_spec(D), _row_spec(D), _row_spec(D), _row_spec(D),
              _row_spec(D), _full_spec((D, D)), _full_spec((1, D)),
              _full_spec((D, D))],
    out_specs=[_row_spec(D)],
    out_shape=[jax.ShapeDtypeStruct((NPAD, D), jnp.float32)],
)

_final = pl.pallas_call(
    _final_body,
    grid=(GRID,),
    in_specs=[_row_spec(D), _row_spec(D), _row_spec(D), _row_spec(D),
              _row_spec(D), _full_spec((D, D)), _full_spec((1, D)),
              _full_spec((D, D)), _full_spec((1, D)), _row_spec(1),
              _full_spec((1, 1))],
    out_specs=[_full_spec((G, D)), _full_spec((G, D)), _full_spec((G, 1))],
    out_shape=[jax.ShapeDtypeStruct((G, D), jnp.float32),
               jax.ShapeDtypeStruct((G, D), jnp.float32),
               jax.ShapeDtypeStruct((G, 1), jnp.float32)],
)


@jax.jit
def kernel(x, edge_index, batch, Wl1, bl1, Wr1, Wl2, bl2, Wr2, Wf, bf):
    src = edge_index[0].astype(jnp.int32)
    dst = edge_index[1].astype(jnp.int32)
    src3 = jnp.pad(src, (0, EPAD - E)).reshape(NWORK * ROWS_W, CHUNK)
    dst3 = jnp.pad(dst, (0, EPAD - E), constant_values=N).reshape(
        NWORK * ROWS_W, CHUNK)
    xp = jnp.pad(x, ((0, NPAD - N), (0, 0)))
    batch_p = jnp.pad(batch.astype(jnp.int32), (0, NPAD - N),
                      constant_values=G).reshape(NPAD, 1)
    wl1t = Wl1.T
    wr1t = Wr1.T
    wl2t = Wl2.T
    wr2t = Wr2.T
    b1 = bl1.reshape(1, D)
    b2 = bl2.reshape(1, D)
    wf_row = Wf.reshape(1, D)
    bf2 = bf.reshape(1, 1)

    (cntp,) = _make_sc_kernel(False)(src3, dst3)
    (aggp1,) = _make_sc_kernel(True)(xp, src3, dst3)
    h1 = _layer(aggp1[:NPAD], aggp1[NPAD:], cntp[:NPAD], cntp[NPAD:],
                xp, wl1t, b1, wr1t)[0]
    (aggp2,) = _make_sc_kernel(True)(h1, src3, dst3)
    _, _, out = _final(aggp2[:NPAD], aggp2[NPAD:], cntp[:NPAD], cntp[NPAD:],
                       h1, wl2t, b2, wr2t, wf_row, batch_p, bf2)
    return out


# speedup vs baseline: 3.4301x; 1.0721x over previous
import functools

import jax
import jax.numpy as jnp
from jax import lax
from jax.experimental import pallas as pl
from jax.experimental.pallas import tpu as pltpu
from jax.experimental.pallas import tpu_sc as plsc

N = 10000
NPAD = 10240
E = 320000
D = 128
G = 64
NWORK = 32
CHUNK = 128
ROWS_W = 80
GS = 8
EPAD = NWORK * ROWS_W * CHUNK
STRIPE = NPAD // 16


def _fill_rows(rows, val):
    v = jnp.full((16,), val, jnp.float32)
    lanes = D // 16

    def body(i, _):
        rows[i // lanes, pl.ds((i % lanes) * 16, 16)] = v
        return 0
    lax.fori_loop(0, CHUNK * lanes, body, 0)


def _zero_stripe(rows, acc_sh, base):
    _fill_rows(rows, 0.0)

    def body(k, _):
        pltpu.sync_copy(rows, acc_sh.at[pl.ds(base + k * CHUNK, CHUNK)])
        return 0
    lax.fori_loop(0, STRIPE // CHUNK, body, 0)


def _copy_stripe_out(rows, acc_sh, base, out, obase):
    def body(k, _):
        pltpu.sync_copy(acc_sh.at[pl.ds(base + k * CHUNK, CHUNK)], rows)
        pltpu.sync_copy(rows, out.at[pl.ds(obase + k * CHUNK, CHUNK)])
        return 0
    lax.fori_loop(0, STRIPE // CHUNK, body, 0)


def _sc_body(with_gather, *refs):
    if with_gather:
        (y_hbm, src_hbm, dst_hbm, agg_out,
         srcv, dstv, rows, rows1, acc_sh, gsem, gsem1) = refs
        bufs = (rows, rows1)
        sems = (gsem, gsem1)
    else:
        (src_hbm, dst_hbm, agg_out,
         srcv, dstv, rows, acc_sh, gsem) = refs

    cid = lax.axis_index("c")
    sid = lax.axis_index("s")
    wid = cid * 16 + sid
    base = sid * STRIPE

    _zero_stripe(rows, acc_sh, base)
    if not with_gather:
        _fill_rows(rows, 1.0)
    plsc.subcore_barrier()

    def group(g, _):
        row0 = wid * ROWS_W + g * GS
        if with_gather:
            pltpu.sync_copy(src_hbm.at[pl.ds(row0, GS)], srcv)
        pltpu.sync_copy(dst_hbm.at[pl.ds(row0, GS)], dstv)

        if with_gather:
            desc = [None] * GS
            desc[0] = pltpu.async_copy(y_hbm.at[srcv.at[0]], bufs[0], sems[0])
            for j in range(GS):
                desc[j].wait()
                if j + 1 < GS:
                    desc[j + 1] = pltpu.async_copy(
                        y_hbm.at[srcv.at[j + 1]],
                        bufs[(j + 1) % 2], sems[(j + 1) % 2])
                pltpu.sync_copy(bufs[j % 2], acc_sh.at[dstv.at[j]], add=True)
        else:
            def step(j, _):
                pltpu.sync_copy(rows, acc_sh.at[dstv.at[j]], add=True)
                return 0
            lax.fori_loop(0, GS, step, 0)
        return 0
    lax.fori_loop(0, ROWS_W // GS, group, 0)

    plsc.subcore_barrier()
    _copy_stripe_out(rows, acc_sh, base, agg_out, cid * NPAD + base)


@functools.lru_cache(maxsize=None)
def _make_sc_kernel(with_gather):
    mesh = plsc.VectorSubcoreMesh(core_axis_name="c", subcore_axis_name="s",
                                  num_cores=2, num_subcores=16)
    scratch = [
        pltpu.VMEM((GS, CHUNK), jnp.int32),
        pltpu.VMEM((GS, CHUNK), jnp.int32),
        pltpu.VMEM((CHUNK, D), jnp.float32),
    ]
    if with_gather:
        scratch.append(pltpu.VMEM((CHUNK, D), jnp.float32))
    scratch.append(pltpu.VMEM_SHARED((NPAD, D), jnp.float32))
    scratch.append(pltpu.SemaphoreType.DMA)
    if with_gather:
        scratch.append(pltpu.SemaphoreType.DMA)
    return pl.kernel(
        functools.partial(_sc_body, with_gather),
        out_type=(jax.ShapeDtypeStruct((2 * NPAD, D), jnp.float32),),
        mesh=mesh,
        scratch_types=tuple(scratch),
    )


BLK = 1024
GRID = NPAD // BLK


def _bdot(a, b):
    return jnp.dot(a.astype(jnp.bfloat16), b.astype(jnp.bfloat16),
                   preferred_element_type=jnp.float32)


def _layer_body(p0, p1, c0, c1, x_ref, wl_ref, bl_ref, wr_ref, h_ref):
    cnt = c0[:, 0:1] + c1[:, 0:1]
    mean = (p0[...] + p1[...]) / jnp.maximum(cnt, 1.0)
    h = _bdot(mean, wl_ref[...]) + bl_ref[...] + _bdot(x_ref[...], wr_ref[...])
    h_ref[...] = jnp.maximum(h, 0.0)


def _final_body(p0, p1, c0, c1, x_ref, wl_ref, bl_ref, wr_ref, wf_ref,
                batch_ref, bf_ref, sums_ref, cnts_ref, out_ref):
    i = pl.program_id(0)
    cnt = c0[:, 0:1] + c1[:, 0:1]
    mean = (p0[...] + p1[...]) / jnp.maximum(cnt, 1.0)
    h = _bdot(mean, wl_ref[...]) + bl_ref[...] + _bdot(x_ref[...], wr_ref[...])
    h = jnp.maximum(h, 0.0)

    gids = lax.broadcasted_iota(jnp.int32, (BLK, G), 1)
    onehot = (batch_ref[...] == gids).astype(jnp.float32)
    dn = (((0,), (0,)), ((), ()))
    s = lax.dot_general(onehot, h, dn, precision='highest',
                        preferred_element_type=jnp.float32)
    c = lax.dot_general(onehot, jnp.ones((BLK, D), jnp.float32), dn,
                        precision='highest',
                        preferred_element_type=jnp.float32)

    @pl.when(i == 0)
    def _():
        sums_ref[...] = jnp.zeros((G, D), jnp.float32)
        cnts_ref[...] = jnp.zeros((G, D), jnp.float32)

    sums_ref[...] += s
    cnts_ref[...] += c

    @pl.when(i == GRID - 1)
    def _():
        pooled = sums_ref[...] / jnp.maximum(cnts_ref[...], 1.0)
        pb = pooled.astype(jnp.bfloat16).astype(jnp.float32)
        wb = wf_ref[...].astype(jnp.bfloat16).astype(jnp.float32)
        out_ref[...] = jnp.sum(pb * wb, axis=1, keepdims=True) + bf_ref[...]


def _row_spec(last):
    return pl.BlockSpec((BLK, last), lambda i: (i, 0))


def _full_spec(shape):
    return pl.BlockSpec(shape, lambda i: (0,) * len(shape))


_layer = pl.pallas_call(
    _layer_body,
    grid=(GRID,),
    in_specs=[_row_spec(D), _row_spec(D), _row_spec(D), _row_spec(D),
              _row_spec(D), _full_spec((D, D)), _full_spec((1, D)),
              _full_spec((D, D))],
    out_specs=[_row_spec(D)],
    out_shape=[jax.ShapeDtypeStruct((NPAD, D), jnp.float32)],
)

_final = pl.pallas_call(
    _final_body,
    grid=(GRID,),
    in_specs=[_row_spec(D), _row_spec(D), _row_spec(D), _row_spec(D),
              _row_spec(D), _full_spec((D, D)), _full_spec((1, D)),
              _full_spec((D, D)), _full_spec((1, D)), _row_spec(1),
              _full_spec((1, 1))],
    out_specs=[_full_spec((G, D)), _full_spec((G, D)), _full_spec((G, 1))],
    out_shape=[jax.ShapeDtypeStruct((G, D), jnp.float32),
               jax.ShapeDtypeStruct((G, D), jnp.float32),
               jax.ShapeDtypeStruct((G, 1), jnp.float32)],
)


@jax.jit
def kernel(x, edge_index, batch, Wl1, bl1, Wr1, Wl2, bl2, Wr2, Wf, bf):
    src = edge_index[0].astype(jnp.int32)
    dst = edge_index[1].astype(jnp.int32)
    src3 = jnp.pad(src, (0, EPAD - E)).reshape(NWORK * ROWS_W, CHUNK)
    dst3 = jnp.pad(dst, (0, EPAD - E), constant_values=N).reshape(
        NWORK * ROWS_W, CHUNK)
    xp = jnp.pad(x, ((0, NPAD - N), (0, 0)))
    batch_p = jnp.pad(batch.astype(jnp.int32), (0, NPAD - N),
                      constant_values=G).reshape(NPAD, 1)
    wl1t = Wl1.T
    wr1t = Wr1.T
    wl2t = Wl2.T
    wr2t = Wr2.T
    b1 = bl1.reshape(1, D)
    b2 = bl2.reshape(1, D)
    wf_row = Wf.reshape(1, D)
    bf2 = bf.reshape(1, 1)

    (cntp,) = _make_sc_kernel(False)(src3, dst3)
    (aggp1,) = _make_sc_kernel(True)(xp, src3, dst3)
    h1 = _layer(aggp1[:NPAD], aggp1[NPAD:], cntp[:NPAD], cntp[NPAD:],
                xp, wl1t, b1, wr1t)[0]
    (aggp2,) = _make_sc_kernel(True)(h1, src3, dst3)
    _, _, out = _final(aggp2[:NPAD], aggp2[NPAD:], cntp[:NPAD], cntp[NPAD:],
                       h1, wl2t, b2, wr2t, wf_row, batch_p, bf2)
    return out
